# baseline (device time: 89401 ns/iter reference)
import jax
import jax.numpy as jnp
from jax import lax
from jax.experimental import pallas as pl
from jax.experimental.pallas import tpu as pltpu

N_DEV = 32
E_LOCAL = 4


def _dim_allreduce(partial):
    n, h = partial.shape
    RY = n // 4
    RZ = RY // 4
    RX = RZ // 2

    def body(p_ref, out_ref,
             ycomm, ystage, zcomm, zstage,
             gycomm, gystage, gzcomm, gzstage,
             xsbuf, xrcomm, gxsbuf, gxrcomm,
             yss, yrs, zss, zrs, gyss, gyrs, gzss, gzrs,
             xss, xrs, gxss, gxrs,
             cr_y, cr_z, cr_gz, cr_gy):
        p = lax.axis_index("i")
        z = p // 8
        r = p % 8
        y = r // 2
        b = r % 2
        x = jnp.where(y % 2 == 0, b, 1 - b)

        def pos_of(xx, yy, zz):
            return zz * 8 + yy * 2 + jnp.where(yy % 2 == 0, xx, 1 - xx)

        y_next = pos_of(x, (y + 1) % 4, z)
        y_prev = pos_of(x, (y + 3) % 4, z)
        z_next = pos_of(x, y, (z + 1) % 4)
        z_prev = pos_of(x, y, (z + 3) % 4)
        x_peer = pos_of(1 - x, y, z)

        barrier_sem = pltpu.get_barrier_semaphore()
        for nbr in (y_next, y_prev, z_next, z_prev, x_peer):
            pl.semaphore_signal(
                barrier_sem, inc=1,
                device_id=(nbr,), device_id_type=pl.DeviceIdType.MESH,
            )
        pl.semaphore_wait(barrier_sem, 5)

        def ring_rs(q, nxt, prv, read_block, write_result,
                    stage, comm, ssem, rsem, credit):
            stage[0] = read_block(q)
            for t in range(3):
                s = t % 2
                if t == 2:
                    pl.semaphore_wait(credit, 1)
                rdma = pltpu.make_async_remote_copy(
                    src_ref=stage.at[s], dst_ref=comm.at[s],
                    send_sem=ssem.at[s], recv_sem=rsem.at[s],
                    device_id=(nxt,), device_id_type=pl.DeviceIdType.MESH)
                rdma.start()
                rdma.wait()
                val = comm[s] + read_block((q - t - 1) % 4)
                if t < 2:
                    stage[(t + 1) % 2] = val
                else:
                    write_result(val)
                if t == 0:
                    pl.semaphore_signal(
                        credit, inc=1,
                        device_id=(prv,), device_id_type=pl.DeviceIdType.MESH)

        def ring_ag(q, nxt, prv, read_out, write_out,
                    stage, comm, ssem, rsem, credit):
            stage[0] = read_out((q + 1) % 4)
            for t in range(3):
                s = t % 2
                if t == 2:
                    pl.semaphore_wait(credit, 1)
                rdma = pltpu.make_async_remote_copy(
                    src_ref=stage.at[s], dst_ref=comm.at[s],
                    send_sem=ssem.at[s], recv_sem=rsem.at[s],
                    device_id=(nxt,), device_id_type=pl.DeviceIdType.MESH)
                rdma.start()
                rdma.wait()
                write_out((q - t) % 4, comm[s])
                if t < 2:
                    stage[(t + 1) % 2] = comm[s]
                if t == 0:
                    pl.semaphore_signal(
                        credit, inc=1,
                        device_id=(prv,), device_id_type=pl.DeviceIdType.MESH)

        def exchange(peer, sbuf, rcomm, ssem, rsem):
            rdma = pltpu.make_async_remote_copy(
                src_ref=sbuf, dst_ref=rcomm, send_sem=ssem, recv_sem=rsem,
                device_id=(peer,), device_id_type=pl.DeviceIdType.MESH)
            rdma.start()
            rdma.wait()

        j_y = (y + 1) % 4
        k_z = (z + 1) % 4
        base_y = j_y * RY
        base_z = base_y + k_z * RZ

        def y_write(v):
            out_ref[pl.ds(base_y, RY), :] = v
        ring_rs(y, y_next, y_prev,
                lambda j: p_ref[pl.ds(j * RY, RY), :], y_write,
                ystage, ycomm, yss, yrs, cr_y)

        def z_write(v):
            out_ref[pl.ds(base_z, RZ), :] = v
        ring_rs(z, z_next, z_prev,
                lambda k: out_ref[pl.ds(base_y + k * RZ, RZ), :], z_write,
                zstage, zcomm, zss, zrs, cr_z)

        xsbuf[:, :] = out_ref[pl.ds(base_z + (1 - x) * RX, RX), :]
        exchange(x_peer, xsbuf, xrcomm, xss, xrs)
        keep = pl.ds(base_z + x * RX, RX)
        out_ref[keep, :] = out_ref[keep, :] + xrcomm[:, :]

        gxsbuf[:, :] = out_ref[keep, :]
        exchange(x_peer, gxsbuf, gxrcomm, gxss, gxrs)
        out_ref[pl.ds(base_z + (1 - x) * RX, RX), :] = gxrcomm[:, :]

        def gz_write(k, v):
            out_ref[pl.ds(base_y + k * RZ, RZ), :] = v
        ring_ag(z, z_next, z_prev,
                lambda k: out_ref[pl.ds(base_y + k * RZ, RZ), :], gz_write,
                gzstage, gzcomm, gzss, gzrs, cr_gz)

        def gy_write(j, v):
            out_ref[pl.ds(j * RY, RY), :] = v
        ring_ag(y, y_next, y_prev,
                lambda j: out_ref[pl.ds(j * RY, RY), :], gy_write,
                gystage, gycomm, gyss, gyrs, cr_gy)

    return pl.pallas_call(
        body,
        out_shape=jax.ShapeDtypeStruct((n, h), jnp.float32),
        in_specs=[pl.BlockSpec(memory_space=pltpu.VMEM)],
        out_specs=pl.BlockSpec(memory_space=pltpu.VMEM),
        scratch_shapes=[
            pltpu.VMEM((2, RY, h), jnp.float32),
            pltpu.VMEM((2, RY, h), jnp.float32),
            pltpu.VMEM((2, RZ, h), jnp.float32),
            pltpu.VMEM((2, RZ, h), jnp.float32),
            pltpu.VMEM((2, RY, h), jnp.float32),
            pltpu.VMEM((2, RY, h), jnp.float32),
            pltpu.VMEM((2, RZ, h), jnp.float32),
            pltpu.VMEM((2, RZ, h), jnp.float32),
            pltpu.VMEM((RX, h), jnp.float32),
            pltpu.VMEM((RX, h), jnp.float32),
            pltpu.VMEM((RX, h), jnp.float32),
            pltpu.VMEM((RX, h), jnp.float32),
            pltpu.SemaphoreType.DMA((2,)),
            pltpu.SemaphoreType.DMA((2,)),
            pltpu.SemaphoreType.DMA((2,)),
            pltpu.SemaphoreType.DMA((2,)),
            pltpu.SemaphoreType.DMA((2,)),
            pltpu.SemaphoreType.DMA((2,)),
            pltpu.SemaphoreType.DMA((2,)),
            pltpu.SemaphoreType.DMA((2,)),
            pltpu.SemaphoreType.DMA,
            pltpu.SemaphoreType.DMA,
            pltpu.SemaphoreType.DMA,
            pltpu.SemaphoreType.DMA,
            pltpu.SemaphoreType.REGULAR,
            pltpu.SemaphoreType.REGULAR,
            pltpu.SemaphoreType.REGULAR,
            pltpu.SemaphoreType.REGULAR,
        ],
        compiler_params=pltpu.CompilerParams(collective_id=0),
    )(partial)


def _dual_allreduce(partial):
    n, h = partial.shape
    RY = n // 4
    RZ = RY // 4
    RX = RZ // 2
    HW = h // 2

    def body(p_ref, out_ref,
             s1Ac, s1As, s1Bc, s1Bs,
             s2Ac, s2As, s2Bc, s2Bs,
             s5Ac, s5As, s5Bc, s5Bs,
             s6Ac, s6As, s6Bc, s6Bs,
             xsA, xrA, xsB, xrB, gxsA, gxrA, gxsB, gxrB,
             s1Ass, s1Ars, s1Bss, s1Brs,
             s2Ass, s2Ars, s2Bss, s2Brs,
             s5Ass, s5Ars, s5Bss, s5Brs,
             s6Ass, s6Ars, s6Bss, s6Brs,
             xssA, xrsA, xssB, xrsB, gxssA, gxrsA, gxssB, gxrsB,
             cr1A, cr1B, cr2A, cr2B, cr5A, cr5B, cr6A, cr6B):
        p = lax.axis_index("i")
        z = p // 8
        r = p % 8
        y = r // 2
        b = r % 2
        x = jnp.where(y % 2 == 0, b, 1 - b)

        def pos_of(xx, yy, zz):
            return zz * 8 + yy * 2 + jnp.where(yy % 2 == 0, xx, 1 - xx)

        y_next = pos_of(x, (y + 1) % 4, z)
        y_prev = pos_of(x, (y + 3) % 4, z)
        z_next = pos_of(x, y, (z + 1) % 4)
        z_prev = pos_of(x, y, (z + 3) % 4)
        x_peer = pos_of(1 - x, y, z)

        barrier_sem = pltpu.get_barrier_semaphore()
        for nbr in (y_next, y_prev, z_next, z_prev, x_peer):
            pl.semaphore_signal(
                barrier_sem, inc=1,
                device_id=(nbr,), device_id_type=pl.DeviceIdType.MESH,
            )
        pl.semaphore_wait(barrier_sem, 5)

        def _rdma(stage, comm, ss, rs, s, nxt):
            return pltpu.make_async_remote_copy(
                src_ref=stage.at[s], dst_ref=comm.at[s],
                send_sem=ss.at[s], recv_sem=rs.at[s],
                device_id=(nxt,), device_id_type=pl.DeviceIdType.MESH)

        def _credit(cr, prv):
            pl.semaphore_signal(
                cr, inc=1, device_id=(prv,), device_id_type=pl.DeviceIdType.MESH)

        def ring_rs2(A, B):
            (qa, nxta, prva, reada, writea, cA, sA, ssA, rsA, crA) = A
            (qb, nxtb, prvb, readb, writeb, cB, sB, ssB, rsB, crB) = B
            sA[0] = reada(qa)
            sB[0] = readb(qb)
            for t in range(3):
                s = t % 2
                if t == 2:
                    pl.semaphore_wait(crA, 1)
                    pl.semaphore_wait(crB, 1)
                ra = _rdma(sA, cA, ssA, rsA, s, nxta)
                rb = _rdma(sB, cB, ssB, rsB, s, nxtb)
                ra.start()
                rb.start()
                ra.wait()
                rb.wait()
                va = cA[s] + reada((qa - t - 1) % 4)
                vb = cB[s] + readb((qb - t - 1) % 4)
                if t < 2:
                    sA[(t + 1) % 2] = va
                    sB[(t + 1) % 2] = vb
                else:
                    writea(va)
                    writeb(vb)
                if t == 0:
                    _credit(crA, prva)
                    _credit(crB, prvb)

        def ring_ag2(A, B):
            (qa, nxta, prva, reada, writea, cA, sA, ssA, rsA, crA) = A
            (qb, nxtb, prvb, readb, writeb, cB, sB, ssB, rsB, crB) = B
            sA[0] = reada((qa + 1) % 4)
            sB[0] = readb((qb + 1) % 4)
            for t in range(3):
                s = t % 2
                if t == 2:
                    pl.semaphore_wait(crA, 1)
                    pl.semaphore_wait(crB, 1)
                ra = _rdma(sA, cA, ssA, rsA, s, nxta)
                rb = _rdma(sB, cB, ssB, rsB, s, nxtb)
                ra.start()
                rb.start()
                ra.wait()
                rb.wait()
                writea((qa - t) % 4, cA[s])
                writeb((qb - t) % 4, cB[s])
                if t < 2:
                    sA[(t + 1) % 2] = cA[s]
                    sB[(t + 1) % 2] = cB[s]
                if t == 0:
                    _credit(crA, prva)
                    _credit(crB, prvb)

        def exchange2(sbufA, rcommA, ssA, rsA, sbufB, rcommB, ssB, rsB):
            ra = pltpu.make_async_remote_copy(
                src_ref=sbufA, dst_ref=rcommA, send_sem=ssA, recv_sem=rsA,
                device_id=(x_peer,), device_id_type=pl.DeviceIdType.MESH)
            rb = pltpu.make_async_remote_copy(
                src_ref=sbufB, dst_ref=rcommB, send_sem=ssB, recv_sem=rsB,
                device_id=(x_peer,), device_id_type=pl.DeviceIdType.MESH)
            ra.start()
            rb.start()
            ra.wait()
            rb.wait()

        L = pl.ds(0, HW)
        Rc = pl.ds(HW, HW)

        j_yA = (y + 1) % 4
        j_zB = (z + 1) % 4
        k_zA = (z + 1) % 4
        k_yB = (y + 1) % 4
        baseA1 = j_yA * RY
        baseB1 = j_zB * RY
        baseA2 = baseA1 + k_zA * RZ
        baseB2 = baseB1 + k_yB * RZ

        def w1A(v):
            out_ref[pl.ds(baseA1, RY), L] = v

        def w1B(v):
            out_ref[pl.ds(baseB1, RY), Rc] = v

        ring_rs2(
            (y, y_next, y_prev, lambda j: p_ref[pl.ds(j * RY, RY), L], w1A,
             s1Ac, s1As, s1Ass, s1Ars, cr1A),
            (z, z_next, z_prev, lambda j: p_ref[pl.ds(j * RY, RY), Rc], w1B,
             s1Bc, s1Bs, s1Bss, s1Brs, cr1B),
        )

        def w2A(v):
            out_ref[pl.ds(baseA2, RZ), L] = v

        def w2B(v):
            out_ref[pl.ds(baseB2, RZ), Rc] = v

        ring_rs2(
            (z, z_next, z_prev,
             lambda k: out_ref[pl.ds(baseA1 + k * RZ, RZ), L], w2A,
             s2Ac, s2As, s2Ass, s2Ars, cr2A),
            (y, y_next, y_prev,
             lambda k: out_ref[pl.ds(baseB1 + k * RZ, RZ), Rc], w2B,
             s2Bc, s2Bs, s2Bss, s2Brs, cr2B),
        )

        xsA[:, :] = out_ref[pl.ds(baseA2 + (1 - x) * RX, RX), L]
        xsB[:, :] = out_ref[pl.ds(baseB2 + (1 - x) * RX, RX), Rc]
        exchange2(xsA, xrA, xssA, xrsA, xsB, xrB, xssB, xrsB)
        keepA = pl.ds(baseA2 + x * RX, RX)
        keepB = pl.ds(baseB2 + x * RX, RX)
        out_ref[keepA, L] = out_ref[keepA, L] + xrA[:, :]
        out_ref[keepB, Rc] = out_ref[keepB, Rc] + xrB[:, :]

        gxsA[:, :] = out_ref[keepA, L]
        gxsB[:, :] = out_ref[keepB, Rc]
        exchange2(gxsA, gxrA, gxssA, gxrsA, gxsB, gxrB, gxssB, gxrsB)
        out_ref[pl.ds(baseA2 + (1 - x) * RX, RX), L] = gxrA[:, :]
        out_ref[pl.ds(baseB2 + (1 - x) * RX, RX), Rc] = gxrB[:, :]

        def g5A(k, v):
            out_ref[pl.ds(baseA1 + k * RZ, RZ), L] = v

        def g5B(k, v):
            out_ref[pl.ds(baseB1 + k * RZ, RZ), Rc] = v

        ring_ag2(
            (z, z_next, z_prev,
             lambda k: out_ref[pl.ds(baseA1 + k * RZ, RZ), L], g5A,
             s5Ac, s5As, s5Ass, s5Ars, cr5A),
            (y, y_next, y_prev,
             lambda k: out_ref[pl.ds(baseB1 + k * RZ, RZ), Rc], g5B,
             s5Bc, s5Bs, s5Bss, s5Brs, cr5B),
        )

        def g6A(j, v):
            out_ref[pl.ds(j * RY, RY), L] = v

        def g6B(j, v):
            out_ref[pl.ds(j * RY, RY), Rc] = v

        ring_ag2(
            (y, y_next, y_prev,
             lambda j: out_ref[pl.ds(j * RY, RY), L], g6A,
             s6Ac, s6As, s6Ass, s6Ars, cr6A),
            (z, z_next, z_prev,
             lambda j: out_ref[pl.ds(j * RY, RY), Rc], g6B,
             s6Bc, s6Bs, s6Bss, s6Brs, cr6B),
        )

    big = [pltpu.VMEM((2, RY, HW), jnp.float32)] * 4
    small = [pltpu.VMEM((2, RZ, HW), jnp.float32)] * 4
    xbuf = [pltpu.VMEM((RX, HW), jnp.float32)] * 8
    sem2 = [pltpu.SemaphoreType.DMA((2,))] * 16
    sem1 = [pltpu.SemaphoreType.DMA] * 8
    creds = [pltpu.SemaphoreType.REGULAR] * 8
    return pl.pallas_call(
        body,
        out_shape=jax.ShapeDtypeStruct((n, h), jnp.float32),
        in_specs=[pl.BlockSpec(memory_space=pltpu.VMEM)],
        out_specs=pl.BlockSpec(memory_space=pltpu.VMEM),
        scratch_shapes=(big + small + small + big + xbuf
                        + sem2 + sem1 + creds),
        compiler_params=pltpu.CompilerParams(collective_id=0),
    )(partial)


def _fused_moe(xin, router_W, route_idx_T, expert_W, shared_W):
    n, d = xin.shape
    h = expert_W.shape[2]
    RY = n // 4
    RZ = RY // 4
    RX = RZ // 2
    HW = h // 2

    def body(x_ref, rw_ref, ri_ref, w_ref, sw_ref, out_ref,
             shared_buf, c_ref,
             s1Ac, s1As, s1Bc, s1Bs,
             s2Ac, s2As, s2Bc, s2Bs,
             s5Ac, s5As, s5Bc, s5Bs,
             s6Ac, s6As, s6Bc, s6Bs,
             xsA, xrA, xsB, xrB,
             s1Ass, s1Ars, s1Bss, s1Brs,
             s2Ass, s2Ars, s2Bss, s2Brs,
             s5Ass, s5Ars, s5Bss, s5Brs,
             s6Ass, s6Ars, s6Bss, s6Brs,
             xssA, xrsA, xssB, xrsB,
             cr1A, cr1B, cr2A, cr2B, cr5A, cr5B, cr6A, cr6B):
        p = lax.axis_index("i")
        z = p // 8
        r = p % 8
        y = r // 2
        bb = r % 2
        xc = jnp.where(y % 2 == 0, bb, 1 - bb)

        def pos_of(xx, yy, zz):
            return zz * 8 + yy * 2 + jnp.where(yy % 2 == 0, xx, 1 - xx)

        y_next = pos_of(xc, (y + 1) % 4, z)
        y_prev = pos_of(xc, (y + 3) % 4, z)
        z_next = pos_of(xc, y, (z + 1) % 4)
        z_prev = pos_of(xc, y, (z + 3) % 4)
        x_peer = pos_of(1 - xc, y, z)

        barrier_sem = pltpu.get_barrier_semaphore()
        for nbr in (y_next, y_prev, z_next, z_prev, x_peer):
            pl.semaphore_signal(
                barrier_sem, inc=1,
                device_id=(nbr,), device_id_type=pl.DeviceIdType.MESH,
            )

        dims = (((1,), (0,)), ((), ()))

        scoresT = lax.dot_general(rw_ref[:, :], x_ref[:, :],
                                  (((0,), (1,)), ((), ())),
                                  preferred_element_type=jnp.float32)
        mx = jnp.max(scoresT, axis=0, keepdims=True)
        ex = jnp.exp(scoresT - mx)
        probsT = ex / jnp.sum(ex, axis=0, keepdims=True)
        e_iota = lax.broadcasted_iota(jnp.int32, (probsT.shape[0], 1), 0)
        for s in range(E_LOCAL):
            e_s = E_LOCAL * p + s
            row = jnp.sum(jnp.where(e_iota == e_s, probsT, 0.0),
                          axis=0, keepdims=True)
            c_ref[pl.ds(s, 1), :] = jnp.where(ri_ref[:, :] == e_s, row, 0.0)

        pl.semaphore_wait(barrier_sem, 5)

        def pblock(j, cs):
            xb = x_ref[pl.ds(j * RY, RY), :]
            acc = None
            for s in range(E_LOCAL):
                coef = c_ref[s, pl.ds(j * RY, RY)]
                t = lax.dot_general(xb * coef[:, None], w_ref[s, :, cs],
                                    dims, preferred_element_type=jnp.float32)
                acc = t if acc is None else acc + t
            return acc

        def shared_block(j):
            shared_buf[pl.ds(j * RY, RY), :] = lax.dot_general(
                x_ref[pl.ds(j * RY, RY), :], sw_ref[:, :],
                dims, preferred_element_type=jnp.float32)

        def _rdma(stage, comm, ss, rs, s, nxt):
            return pltpu.make_async_remote_copy(
                src_ref=stage.at[s], dst_ref=comm.at[s],
                send_sem=ss.at[s], recv_sem=rs.at[s],
                device_id=(nxt,), device_id_type=pl.DeviceIdType.MESH)

        def _credit(cr, prv):
            pl.semaphore_signal(
                cr, inc=1, device_id=(prv,), device_id_type=pl.DeviceIdType.MESH)

        def ring_rs2(A, B, local_a, local_b, extra=None):
            (qa, nxta, prva, writea, cA, sA, ssA, rsA, crA) = A
            (qb, nxtb, prvb, writeb, cB, sB, ssB, rsB, crB) = B
            for t in range(3):
                s = t % 2
                if t == 2:
                    pl.semaphore_wait(crA, 1)
                    pl.semaphore_wait(crB, 1)
                if t == 0:
                    sA[0] = local_a(qa)
                ra = _rdma(sA, cA, ssA, rsA, s, nxta)
                ra.start()
                if t == 0:
                    sB[0] = local_b(qb)
                rb = _rdma(sB, cB, ssB, rsB, s, nxtb)
                rb.start()
                la = local_a((qa - t - 1) % 4)
                lb = local_b((qb - t - 1) % 4)
                if extra is not None:
                    extra(t)
                ra.wait()
                rb.wait()
                va = cA[s] + la
                vb = cB[s] + lb
                if t < 2:
                    sA[(t + 1) % 2] = va
                    sB[(t + 1) % 2] = vb
                else:
                    writea(va)
                    writeb(vb)
                if t == 0:
                    _credit(crA, prva)
                    _credit(crB, prvb)

        def ring_ag2(A, B):
            (qa, nxta, prva, reada, writea, cA, sA, ssA, rsA, crA) = A
            (qb, nxtb, prvb, readb, writeb, cB, sB, ssB, rsB, crB) = B
            sA[0] = reada((qa + 1) % 4)
            sB[0] = readb((qb + 1) % 4)
            for t in range(3):
                s = t % 2
                if t == 2:
                    pl.semaphore_wait(crA, 1)
                    pl.semaphore_wait(crB, 1)
                ra = _rdma(sA, cA, ssA, rsA, s, nxta)
                rb = _rdma(sB, cB, ssB, rsB, s, nxtb)
                ra.start()
                rb.start()
                ra.wait()
                rb.wait()
                writea((qa - t) % 4, cA[s])
                writeb((qb - t) % 4, cB[s])
                if t < 2:
                    sA[(t + 1) % 2] = cA[s]
                    sB[(t + 1) % 2] = cB[s]
                if t == 0:
                    _credit(crA, prva)
                    _credit(crB, prvb)

        def exchange2(sbufA, rcommA, ssA, rsA, sbufB, rcommB, ssB, rsB,
                      extra=None):
            ra = pltpu.make_async_remote_copy(
                src_ref=sbufA, dst_ref=rcommA, send_sem=ssA, recv_sem=rsA,
                device_id=(x_peer,), device_id_type=pl.DeviceIdType.MESH)
            rb = pltpu.make_async_remote_copy(
                src_ref=sbufB, dst_ref=rcommB, send_sem=ssB, recv_sem=rsB,
                device_id=(x_peer,), device_id_type=pl.DeviceIdType.MESH)
            ra.start()
            rb.start()
            if extra is not None:
                extra()
            ra.wait()
            rb.wait()

        L = pl.ds(0, HW)
        Rc = pl.ds(HW, HW)

        j_yA = (y + 1) % 4
        j_zB = (z + 1) % 4
        k_zA = (z + 1) % 4
        k_yB = (y + 1) % 4
        baseA1 = j_yA * RY
        baseB1 = j_zB * RY
        baseA2 = baseA1 + k_zA * RZ
        baseB2 = baseB1 + k_yB * RZ

        def w1A(v):
            out_ref[pl.ds(baseA1, RY), L] = v

        def w1B(v):
            out_ref[pl.ds(baseB1, RY), Rc] = v

        ring_rs2(
            (y, y_next, y_prev, w1A, s1Ac, s1As, s1Ass, s1Ars, cr1A),
            (z, z_next, z_prev, w1B, s1Bc, s1Bs, s1Bss, s1Brs, cr1B),
            lambda j: pblock(j, L), lambda j: pblock(j, Rc),
        )

        def w2A(v):
            out_ref[pl.ds(baseA2, RZ), L] = v

        def w2B(v):
            out_ref[pl.ds(baseB2, RZ), Rc] = v

        def loc2A(k):
            return out_ref[pl.ds(baseA1 + k * RZ, RZ), L]

        def loc2B(k):
            return out_ref[pl.ds(baseB1 + k * RZ, RZ), Rc]

        ring_rs2(
            (z, z_next, z_prev, w2A, s2Ac, s2As, s2Ass, s2Ars, cr2A),
            (y, y_next, y_prev, w2B, s2Bc, s2Bs, s2Bss, s2Brs, cr2B),
            loc2A, loc2B, extra=shared_block,
        )

        blkA = pl.ds(baseA2, RZ)
        blkB = pl.ds(baseB2, RZ)
        xsA[:, :] = out_ref[blkA, L]
        xsB[:, :] = out_ref[blkB, Rc]
        exchange2(xsA, xrA, xssA, xrsA, xsB, xrB, xssB, xrsB,
                  extra=lambda: shared_block(3))
        out_ref[blkA, L] = out_ref[blkA, L] + xrA[:, :]
        out_ref[blkB, Rc] = out_ref[blkB, Rc] + xrB[:, :]

        def g5A(k, v):
            out_ref[pl.ds(baseA1 + k * RZ, RZ), L] = v

        def g5B(k, v):
            out_ref[pl.ds(baseB1 + k * RZ, RZ), Rc] = v

        ring_ag2(
            (z, z_next, z_prev, loc2A, g5A, s5Ac, s5As, s5Ass, s5Ars, cr5A),
            (y, y_next, y_prev, loc2B, g5B, s5Bc, s5Bs, s5Bss, s5Brs, cr5B),
        )

        def g6A(j, v):
            rows = pl.ds(j * RY, RY)
            out_ref[rows, L] = v + shared_buf[rows, L]

        def g6B(j, v):
            rows = pl.ds(j * RY, RY)
            out_ref[rows, Rc] = v + shared_buf[rows, Rc]

        ring_ag2(
            (y, y_next, y_prev,
             lambda j: out_ref[pl.ds(j * RY, RY), L], g6A,
             s6Ac, s6As, s6Ass, s6Ars, cr6A),
            (z, z_next, z_prev,
             lambda j: out_ref[pl.ds(j * RY, RY), Rc], g6B,
             s6Bc, s6Bs, s6Bss, s6Brs, cr6B),
        )
        ownA = pl.ds(baseA1, RY)
        ownB = pl.ds(baseB1, RY)
        out_ref[ownA, L] = out_ref[ownA, L] + shared_buf[ownA, L]
        out_ref[ownB, Rc] = out_ref[ownB, Rc] + shared_buf[ownB, Rc]

    big = [pltpu.VMEM((2, RY, HW), jnp.float32)] * 4
    small = [pltpu.VMEM((2, RZ, HW), jnp.float32)] * 4
    xbuf = [pltpu.VMEM((RZ, HW), jnp.float32)] * 4
    sem2 = [pltpu.SemaphoreType.DMA((2,))] * 16
    sem1 = [pltpu.SemaphoreType.DMA] * 4
    creds = [pltpu.SemaphoreType.REGULAR] * 8
    return pl.pallas_call(
        body,
        out_shape=jax.ShapeDtypeStruct((n, h), jnp.float32),
        in_specs=[pl.BlockSpec(memory_space=pltpu.VMEM)] * 5,
        out_specs=pl.BlockSpec(memory_space=pltpu.VMEM),
        scratch_shapes=([pltpu.VMEM((n, h), jnp.float32),
                         pltpu.VMEM((E_LOCAL, n), jnp.float32)]
                        + big + small + small + big + xbuf
                        + sem2 + sem1 + creds),
        compiler_params=pltpu.CompilerParams(collective_id=0),
    )(xin, router_W, route_idx_T, expert_W, shared_W)


def kernel(x, router_W, route_idx, expert_W, shared_W):
    return _fused_moe(x, router_W, route_idx.T, expert_W, shared_W)


# device time: 84940 ns/iter; 1.0525x vs baseline; 1.0525x over previous
import jax
import jax.numpy as jnp
from jax import lax
from jax.experimental import pallas as pl
from jax.experimental.pallas import tpu as pltpu

N_DEV = 32
E_LOCAL = 4


def _fused_moe(xin, router_W, route_idx_T, expert_W, shared_W):
    n, d = xin.shape
    h = expert_W.shape[2]
    RY = n // 4
    RZ = RY // 4
    HW = h // 2
    HQ = HW // 2

    def body(x_ref, rw_ref, ri_ref, w_ref, sw_ref, out_ref,
             shared_buf, c_ref,
             q1c0, q1t0, q1c1, q1t1, q1c2, q1t2, q1c3, q1t3,
             q2c0, q2t0, q2c1, q2t1,
             q5c0, q5t0, q5c1, q5t1,
             q6c0, q6t0, q6c1, q6t1, q6c2, q6t2, q6c3, q6t3,
             xsA, xrA, xsB, xrB,
             q1ss0, q1rs0, q1ss1, q1rs1, q1ss2, q1rs2, q1ss3, q1rs3,
             q2ss0, q2rs0, q2ss1, q2rs1,
             q5ss0, q5rs0, q5ss1, q5rs1,
             q6ss0, q6rs0, q6ss1, q6rs1, q6ss2, q6rs2, q6ss3, q6rs3,
             xssA, xrsA, xssB, xrsB,
             q1r0, q1r1, q1r2, q1r3, q2r0, q2r1,
             q5r0, q5r1, q6r0, q6r1, q6r2, q6r3):
        p = lax.axis_index("i")
        z = p // 8
        r = p % 8
        y = r // 2
        bb = r % 2
        xc = jnp.where(y % 2 == 0, bb, 1 - bb)

        def pos_of(xx, yy, zz):
            return zz * 8 + yy * 2 + jnp.where(yy % 2 == 0, xx, 1 - xx)

        y_next = pos_of(xc, (y + 1) % 4, z)
        y_prev = pos_of(xc, (y + 3) % 4, z)
        z_next = pos_of(xc, y, (z + 1) % 4)
        z_prev = pos_of(xc, y, (z + 3) % 4)
        x_peer = pos_of(1 - xc, y, z)

        barrier_sem = pltpu.get_barrier_semaphore()
        for nbr in (y_next, y_prev, z_next, z_prev, x_peer):
            pl.semaphore_signal(
                barrier_sem, inc=1,
                device_id=(nbr,), device_id_type=pl.DeviceIdType.MESH,
            )

        dims = (((1,), (0,)), ((), ()))

        scoresT = lax.dot_general(rw_ref[:, :], x_ref[:, :],
                                  (((0,), (1,)), ((), ())),
                                  preferred_element_type=jnp.float32)
        mx = jnp.max(scoresT, axis=0, keepdims=True)
        ex = jnp.exp(scoresT - mx)
        probsT = ex / jnp.sum(ex, axis=0, keepdims=True)
        e_iota = lax.broadcasted_iota(jnp.int32, (probsT.shape[0], 1), 0)
        for s in range(E_LOCAL):
            e_s = E_LOCAL * p + s
            row = jnp.sum(jnp.where(e_iota == e_s, probsT, 0.0),
                          axis=0, keepdims=True)
            c_ref[pl.ds(s, 1), :] = jnp.where(ri_ref[:, :] == e_s, row, 0.0)

        pl.semaphore_wait(barrier_sem, 5)

        def pblock(j, cs):
            xb = x_ref[pl.ds(j * RY, RY), :]
            acc = None
            for s in range(E_LOCAL):
                coef = c_ref[s, pl.ds(j * RY, RY)]
                t = lax.dot_general(xb * coef[:, None], w_ref[s, :, cs],
                                    dims, preferred_element_type=jnp.float32)
                acc = t if acc is None else acc + t
            return acc

        def shared_block(j):
            shared_buf[pl.ds(j * RY, RY), :] = lax.dot_general(
                x_ref[pl.ds(j * RY, RY), :], sw_ref[:, :],
                dims, preferred_element_type=jnp.float32)

        def _rdma(stage, comm, ss, rs, s, nxt):
            return pltpu.make_async_remote_copy(
                src_ref=stage.at[s], dst_ref=comm.at[s],
                send_sem=ss.at[s], recv_sem=rs.at[s],
                device_id=(nxt,), device_id_type=pl.DeviceIdType.MESH)

        def _credit(cr, prv):
            pl.semaphore_signal(
                cr, inc=1, device_id=(prv,), device_id_type=pl.DeviceIdType.MESH)

        def ring_rs_pipe(chains, extra=None):
            rd = [None] * len(chains)
            for i, (q, nxt, prv, local, write, cm, st, ss, rs, cr) in (
                    enumerate(chains)):
                st[0] = local(q)
                rd[i] = _rdma(st, cm, ss, rs, 0, nxt)
                rd[i].start()
            for t in range(3):
                s = t % 2
                pend = [ch[3]((ch[0] - t - 1) % 4) for ch in chains]
                if extra is not None:
                    extra(t)
                for i, (q, nxt, prv, local, write, cm, st, ss, rs, cr) in (
                        enumerate(chains)):
                    rd[i].wait()
                    val = cm[s] + pend[i]
                    if t < 2:
                        st[(t + 1) % 2] = val
                        if t == 1:
                            pl.semaphore_wait(cr, 1)
                        rd[i] = _rdma(st, cm, ss, rs, (t + 1) % 2, nxt)
                        rd[i].start()
                    else:
                        write(val)
                    if t == 0:
                        _credit(cr, prv)

        def ring_ag_pipe(chains):
            rd = [None] * len(chains)
            for i, (q, nxt, prv, read, write, cm, st, ss, rs, cr) in (
                    enumerate(chains)):
                st[0] = read((q + 1) % 4)
                rd[i] = _rdma(st, cm, ss, rs, 0, nxt)
                rd[i].start()
            for t in range(3):
                s = t % 2
                for i, (q, nxt, prv, read, write, cm, st, ss, rs, cr) in (
                        enumerate(chains)):
                    rd[i].wait()
                    if t < 2:
                        st[(t + 1) % 2] = cm[s]
                        if t == 1:
                            pl.semaphore_wait(cr, 1)
                        rd[i] = _rdma(st, cm, ss, rs, (t + 1) % 2, nxt)
                        rd[i].start()
                    write((q - t) % 4, cm[s])
                    if t == 0:
                        _credit(cr, prv)

        def exchange2(sbufA, rcommA, ssA, rsA, sbufB, rcommB, ssB, rsB,
                      extra=None):
            ra = pltpu.make_async_remote_copy(
                src_ref=sbufA, dst_ref=rcommA, send_sem=ssA, recv_sem=rsA,
                device_id=(x_peer,), device_id_type=pl.DeviceIdType.MESH)
            rb = pltpu.make_async_remote_copy(
                src_ref=sbufB, dst_ref=rcommB, send_sem=ssB, recv_sem=rsB,
                device_id=(x_peer,), device_id_type=pl.DeviceIdType.MESH)
            ra.start()
            rb.start()
            if extra is not None:
                extra()
            ra.wait()
            rb.wait()

        L = pl.ds(0, HW)
        Rc = pl.ds(HW, HW)
        Q = [pl.ds(i * HQ, HQ) for i in range(4)]

        j_yA = (y + 1) % 4
        j_zB = (z + 1) % 4
        k_zA = (z + 1) % 4
        k_yB = (y + 1) % 4
        baseA1 = j_yA * RY
        baseB1 = j_zB * RY
        baseA2 = baseA1 + k_zA * RZ
        baseB2 = baseB1 + k_yB * RZ

        def mkw1(base, cs):
            def w(v):
                out_ref[pl.ds(base, RY), cs] = v
            return w

        ring_rs_pipe([
            (y, y_next, y_prev, lambda j: pblock(j, Q[0]),
             mkw1(baseA1, Q[0]), q1c0, q1t0, q1ss0, q1rs0, q1r0),
            (y, y_next, y_prev, lambda j: pblock(j, Q[1]),
             mkw1(baseA1, Q[1]), q1c1, q1t1, q1ss1, q1rs1, q1r1),
            (z, z_next, z_prev, lambda j: pblock(j, Q[2]),
             mkw1(baseB1, Q[2]), q1c2, q1t2, q1ss2, q1rs2, q1r2),
            (z, z_next, z_prev, lambda j: pblock(j, Q[3]),
             mkw1(baseB1, Q[3]), q1c3, q1t3, q1ss3, q1rs3, q1r3),
        ])

        def loc2A(k):
            return out_ref[pl.ds(baseA1 + k * RZ, RZ), L]

        def loc2B(k):
            return out_ref[pl.ds(baseB1 + k * RZ, RZ), Rc]

        def w2A(v):
            out_ref[pl.ds(baseA2, RZ), L] = v

        def w2B(v):
            out_ref[pl.ds(baseB2, RZ), Rc] = v

        ring_rs_pipe([
            (z, z_next, z_prev, loc2A, w2A, q2c0, q2t0, q2ss0, q2rs0, q2r0),
            (y, y_next, y_prev, loc2B, w2B, q2c1, q2t1, q2ss1, q2rs1, q2r1),
        ], extra=shared_block)

        blkA = pl.ds(baseA2, RZ)
        blkB = pl.ds(baseB2, RZ)
        xsA[:, :] = out_ref[blkA, L]
        xsB[:, :] = out_ref[blkB, Rc]
        exchange2(xsA, xrA, xssA, xrsA, xsB, xrB, xssB, xrsB,
                  extra=lambda: shared_block(3))
        out_ref[blkA, L] = out_ref[blkA, L] + xrA[:, :]
        out_ref[blkB, Rc] = out_ref[blkB, Rc] + xrB[:, :]

        def g5A(k, v):
            out_ref[pl.ds(baseA1 + k * RZ, RZ), L] = v

        def g5B(k, v):
            out_ref[pl.ds(baseB1 + k * RZ, RZ), Rc] = v

        ring_ag_pipe([
            (z, z_next, z_prev, loc2A, g5A, q5c0, q5t0, q5ss0, q5rs0, q5r0),
            (y, y_next, y_prev, loc2B, g5B, q5c1, q5t1, q5ss1, q5rs1, q5r1),
        ])

        def mkr6(cs):
            def rd6(j):
                return out_ref[pl.ds(j * RY, RY), cs]
            return rd6

        def mkw6(cs):
            def w(j, v):
                rows = pl.ds(j * RY, RY)
                out_ref[rows, cs] = v + shared_buf[rows, cs]
            return w

        ring_ag_pipe([
            (y, y_next, y_prev, mkr6(Q[0]), mkw6(Q[0]),
             q6c0, q6t0, q6ss0, q6rs0, q6r0),
            (y, y_next, y_prev, mkr6(Q[1]), mkw6(Q[1]),
             q6c1, q6t1, q6ss1, q6rs1, q6r1),
            (z, z_next, z_prev, mkr6(Q[2]), mkw6(Q[2]),
             q6c2, q6t2, q6ss2, q6rs2, q6r2),
            (z, z_next, z_prev, mkr6(Q[3]), mkw6(Q[3]),
             q6c3, q6t3, q6ss3, q6rs3, q6r3),
        ])
        ownA = pl.ds(baseA1, RY)
        ownB = pl.ds(baseB1, RY)
        out_ref[ownA, L] = out_ref[ownA, L] + shared_buf[ownA, L]
        out_ref[ownB, Rc] = out_ref[ownB, Rc] + shared_buf[ownB, Rc]

    quar = [pltpu.VMEM((2, RY, HQ), jnp.float32)] * 8
    half = [pltpu.VMEM((2, RZ, HW), jnp.float32)] * 4
    xbuf = [pltpu.VMEM((RZ, HW), jnp.float32)] * 4
    sem2 = [pltpu.SemaphoreType.DMA((2,))] * 24
    sem1 = [pltpu.SemaphoreType.DMA] * 4
    creds = [pltpu.SemaphoreType.REGULAR] * 12
    return pl.pallas_call(
        body,
        out_shape=jax.ShapeDtypeStruct((n, h), jnp.float32),
        in_specs=[pl.BlockSpec(memory_space=pltpu.VMEM)] * 5,
        out_specs=pl.BlockSpec(memory_space=pltpu.VMEM),
        scratch_shapes=([pltpu.VMEM((n, h), jnp.float32),
                         pltpu.VMEM((E_LOCAL, n), jnp.float32)]
                        + quar + half + half + quar + xbuf
                        + sem2 + sem1 + creds),
        compiler_params=pltpu.CompilerParams(collective_id=0),
    )(xin, router_W, route_idx_T, expert_W, shared_W)


def kernel(x, router_W, route_idx, expert_W, shared_W):
    return _fused_moe(x, router_W, route_idx.T, expert_W, shared_W)
